# baseline (speedup 1.0000x reference)
import jax
import jax.numpy as jnp
from jax.experimental import pallas as pl

HID = 16
NUM_LAYERS = 4
TOP_K = 2
MAX_ROUTING = 2
SCORE_SCALE_C = 1.0
BATCH = 256
D_FLAT = HID * 16 * 16

F32 = jnp.float32


def _conv1_kernel(p_ref, w_ref, b_ref, o_ref):
    o_ref[...] = jax.nn.relu(
        jnp.dot(p_ref[...], w_ref[...], preferred_element_type=F32) + b_ref[...])


def _gate_kernel(uf_ref, gate_ref, gb_ref, wes_ref, lg_ref):
    logits = jnp.dot(uf_ref[...], gate_ref[...],
                     preferred_element_type=F32) + gb_ref[...]
    lg_ref[...] = logits
    m = jnp.max(logits, axis=1, keepdims=True)
    ew = jnp.exp(logits - m)
    w = ew / jnp.sum(ew, axis=1, keepdims=True)
    cols = [w[:, e:e + 1] for e in range(NUM_LAYERS)]
    sels = []
    for e in range(NUM_LAYERS):
        rank = jnp.zeros_like(cols[e])
        for j in range(NUM_LAYERS):
            if j == e:
                continue
            beats = (cols[j] > cols[e]) | ((cols[j] == cols[e]) & (j < e))
            rank = rank + jnp.where(beats, 1.0, 0.0)
        sels.append(rank < TOP_K)
    tsum = sum(jnp.where(sels[e], cols[e], 0.0) for e in range(NUM_LAYERS))
    wes_ref[...] = jnp.concatenate(
        [jnp.where(sels[e], cols[e], 0.0) / tsum * SCORE_SCALE_C
         for e in range(NUM_LAYERS)], axis=1)


def _expert_kernel(p_ref, wem_ref, be_ref, wx_ref, o_ref):
    eo = jnp.dot(p_ref[...], wem_ref[...],
                 preferred_element_type=F32) + be_ref[...]
    eo = jax.nn.relu(eo)
    wx = wx_ref[...]
    acc = eo[:, 0:HID] * wx[:, 0:1]
    for e in range(1, NUM_LAYERS):
        acc = acc + eo[:, e * HID:(e + 1) * HID] * wx[:, e:e + 1]
    o_ref[...] = acc


def _mmt_kernel(w_ref, a_ref, b_ref, o_ref):
    o_ref[...] = jnp.dot(w_ref[...], a_ref[...],
                         preferred_element_type=F32) + b_ref[...]


def _tail_kernel(p_ref, w_ref, b_ref, o_ref):
    o = jax.nn.relu(jnp.dot(p_ref[...], w_ref[...],
                            preferred_element_type=F32) + b_ref[...])
    o = o.reshape(BATCH * 16, 4, HID // 2)
    o_ref[...] = jnp.max(o, axis=1)


def _head_kernel(h_ref, w1_ref, b1_ref, w2_ref, b2_ref, o_ref):
    h = jax.nn.relu(jnp.dot(h_ref[...], w1_ref[...],
                            preferred_element_type=F32) + b1_ref[...])
    o_ref[...] = jnp.dot(h, w2_ref[...],
                         preferred_element_type=F32) + b2_ref[...]


def _im2col_s1(u4):
    up = jnp.pad(u4, ((0, 0), (1, 1), (1, 1), (0, 0)))
    parts = [up[:, dy:dy + 16, dx:dx + 16, :]
             for dy in range(3) for dx in range(3)]
    return jnp.concatenate(parts, axis=-1).reshape(BATCH * 256, 9 * HID)


def kernel(x, W1, b1, We, be, Wl, bl, fc1w, fc1b, fc2w, fc2b,
           gate_w, gate_b, tw, tb):
    w1m = W1.reshape(HID, 27).T
    wem = jnp.transpose(We, (3, 4, 2, 0, 1)).reshape(9 * HID, NUM_LAYERS * HID)
    bef = be.reshape(1, NUM_LAYERS * HID)
    wlm = jnp.transpose(Wl, (2, 3, 1, 0)).reshape(9 * HID, HID // 2)
    gtp = gate_w.T
    tbc = tb[:, None]

    xp = jnp.pad(x, ((0, 0), (0, 0), (1, 1), (1, 1)))
    sl = [xp[:, :, ky:ky + 32:2, kx:kx + 32:2]
          for ky in range(3) for kx in range(3)]
    pat = jnp.stack(sl, axis=2)
    pat = pat.transpose(0, 3, 4, 1, 2).reshape(BATCH * 256, 27)
    rb = BATCH * 256 // 8
    u = pl.pallas_call(
        _conv1_kernel,
        grid=(8,),
        in_specs=[pl.BlockSpec((rb, 27), lambda i: (i, 0)),
                  pl.BlockSpec((27, HID), lambda i: (0, 0)),
                  pl.BlockSpec((1, HID), lambda i: (0, 0))],
        out_specs=pl.BlockSpec((rb, HID), lambda i: (i, 0)),
        out_shape=jax.ShapeDtypeStruct((BATCH * 256, HID), F32),
    )(pat, w1m, b1[None, :])

    u4 = u.reshape(BATCH, 16, 16, HID)

    logits_out = []
    for _ in range(MAX_ROUTING):
        uf = u4.transpose(0, 3, 1, 2).reshape(BATCH, D_FLAT)
        wes, lg = pl.pallas_call(
            _gate_kernel,
            in_specs=[pl.BlockSpec((BATCH, D_FLAT), lambda: (0, 0)),
                      pl.BlockSpec((D_FLAT, NUM_LAYERS), lambda: (0, 0)),
                      pl.BlockSpec((1, NUM_LAYERS), lambda: (0, 0))],
            out_specs=[pl.BlockSpec((BATCH, NUM_LAYERS), lambda: (0, 0)),
                       pl.BlockSpec((BATCH, NUM_LAYERS), lambda: (0, 0))],
            out_shape=[jax.ShapeDtypeStruct((BATCH, NUM_LAYERS), F32),
                       jax.ShapeDtypeStruct((BATCH, NUM_LAYERS), F32)],
        )(uf, gtp, gate_b[None, :])
        logits_out.append(lg)
        pat_e = _im2col_s1(u4)
        wx = jnp.broadcast_to(wes[:, None, :], (BATCH, 256, NUM_LAYERS))
        wx = wx.reshape(BATCH * 256, NUM_LAYERS)
        erb = BATCH * 256 // 8
        fin = pl.pallas_call(
            _expert_kernel,
            grid=(8,),
            in_specs=[pl.BlockSpec((erb, 9 * HID), lambda i: (i, 0)),
                      pl.BlockSpec((9 * HID, NUM_LAYERS * HID),
                                   lambda i: (0, 0)),
                      pl.BlockSpec((1, NUM_LAYERS * HID), lambda i: (0, 0)),
                      pl.BlockSpec((erb, NUM_LAYERS), lambda i: (i, 0))],
            out_specs=pl.BlockSpec((erb, HID), lambda i: (i, 0)),
            out_shape=jax.ShapeDtypeStruct((BATCH * 256, HID), F32),
        )(pat_e, wem, bef, wx)
        fint = fin.reshape(BATCH, 16, 16, HID).transpose(3, 1, 2, 0)
        fint = fint.reshape(D_FLAT, BATCH)
        rbk = 512
        outt = pl.pallas_call(
            _mmt_kernel,
            grid=(D_FLAT // rbk,),
            in_specs=[pl.BlockSpec((rbk, D_FLAT), lambda j: (j, 0)),
                      pl.BlockSpec((D_FLAT, BATCH), lambda j: (0, 0)),
                      pl.BlockSpec((rbk, 1), lambda j: (j, 0))],
            out_specs=pl.BlockSpec((rbk, BATCH), lambda j: (j, 0)),
            out_shape=jax.ShapeDtypeStruct((D_FLAT, BATCH), F32),
        )(tw, fint, tbc)
        u4 = outt.reshape(HID, 16, 16, BATCH).transpose(3, 1, 2, 0)

    up = jnp.pad(u4, ((0, 0), (1, 1), (1, 1), (0, 0)))
    sl = [up[:, ky:ky + 16:2, kx:kx + 16:2, :]
          for ky in range(3) for kx in range(3)]
    patl = jnp.concatenate(sl, axis=-1)
    patl = patl.reshape(BATCH, 4, 2, 4, 2, 9 * HID)
    patl = patl.transpose(0, 1, 3, 2, 4, 5).reshape(BATCH * 64, 9 * HID)
    pooled = pl.pallas_call(
        _tail_kernel,
        in_specs=[pl.BlockSpec((BATCH * 64, 9 * HID), lambda: (0, 0)),
                  pl.BlockSpec((9 * HID, HID // 2), lambda: (0, 0)),
                  pl.BlockSpec((1, HID // 2), lambda: (0, 0))],
        out_specs=pl.BlockSpec((BATCH * 16, HID // 2), lambda: (0, 0)),
        out_shape=jax.ShapeDtypeStruct((BATCH * 16, HID // 2), F32),
    )(patl, wlm, bl[None, :])

    h0 = pooled.reshape(BATCH, 16, HID // 2).transpose(0, 2, 1)
    h0 = h0.reshape(BATCH, 128)
    out = pl.pallas_call(
        _head_kernel,
        in_specs=[pl.BlockSpec((BATCH, 128), lambda: (0, 0)),
                  pl.BlockSpec((128, HID), lambda: (0, 0)),
                  pl.BlockSpec((1, HID), lambda: (0, 0)),
                  pl.BlockSpec((HID, 10), lambda: (0, 0)),
                  pl.BlockSpec((1, 10), lambda: (0, 0))],
        out_specs=pl.BlockSpec((BATCH, 10), lambda: (0, 0)),
        out_shape=jax.ShapeDtypeStruct((BATCH, 10), F32),
    )(h0, fc1w.T, fc1b[None, :], fc2w.T, fc2b[None, :])

    return (out, (logits_out[0], logits_out[1]))


# speedup vs baseline: 3.4837x; 3.4837x over previous
import jax
import jax.numpy as jnp
from jax.experimental import pallas as pl

HID = 16
NUM_LAYERS = 4
TOP_K = 2
MAX_ROUTING = 2
SCORE_SCALE_C = 1.0
BATCH = 256
D_FLAT = HID * 16 * 16

F32 = jnp.float32


def _conv1_kernel(w_ref, p_ref, b_ref, o_ref):
    o_ref[...] = jax.nn.relu(
        jnp.dot(w_ref[...], p_ref[...], preferred_element_type=F32) + b_ref[...])


def _gate_kernel(x_ref, gw_ref, gb_ref, o_ref):
    lg = jnp.dot(gw_ref[...], x_ref[...],
                 preferred_element_type=F32) + gb_ref[...]
    rows = [lg[e:e + 1, :] for e in range(NUM_LAYERS)]
    m = jnp.maximum(jnp.maximum(rows[0], rows[1]),
                    jnp.maximum(rows[2], rows[3]))
    ew = [jnp.exp(r - m) for r in rows]
    s = ew[0] + ew[1] + ew[2] + ew[3]
    w = [e / s for e in ew]
    sels = []
    for e in range(NUM_LAYERS):
        rank = jnp.zeros_like(w[e])
        for j in range(NUM_LAYERS):
            if j == e:
                continue
            beats = (w[j] > w[e]) | ((w[j] == w[e]) & (j < e))
            rank = rank + jnp.where(beats, 1.0, 0.0)
        sels.append(rank < TOP_K)
    tsum = sum(jnp.where(sels[e], w[e], 0.0) for e in range(NUM_LAYERS))
    wes = [jnp.where(sels[e], w[e], 0.0) / tsum * SCORE_SCALE_C
           for e in range(NUM_LAYERS)]
    o_ref[...] = jnp.concatenate(wes + rows, axis=0)


def _expert_kernel(xp_ref, wem_ref, be_ref, ws_ref, o_ref):
    i = pl.program_id(0)
    parts = []
    for ky in range(3):
        for kx in range(3):
            parts.append(xp_ref[:, pl.ds(i * 4 + ky, 4), kx:kx + 16, :])
    p = jnp.concatenate(parts, axis=0)
    p = p.reshape(9 * HID, 4 * 16 * BATCH)
    eo = jnp.dot(wem_ref[...], p, preferred_element_type=F32) + be_ref[...]
    eo = jax.nn.relu(eo)
    acc = None
    for e in range(NUM_LAYERS):
        we = ws_ref[e:e + 1, :]
        web = jnp.broadcast_to(we.reshape(1, 1, BATCH), (1, 64, BATCH))
        web = web.reshape(1, 64 * BATCH)
        term = eo[e * HID:(e + 1) * HID, :] * web
        acc = term if acc is None else acc + term
    o_ref[...] = acc


def _mmt_kernel(w_ref, a_ref, b_ref, o_ref):
    o_ref[...] = jnp.dot(w_ref[...], a_ref[...],
                         preferred_element_type=F32) + b_ref[...]


def _tail_kernel(p_ref, wl_ref, bl_ref, w1_ref, b1_ref, w2_ref, b2_ref, o_ref):
    co = jax.nn.relu(jnp.dot(wl_ref[...], p_ref[...],
                             preferred_element_type=F32) + bl_ref[...])
    co = co.reshape(HID // 2, 16, 4, BATCH)
    pooled = jnp.max(co, axis=2)
    pooled = pooled.reshape(128, BATCH)
    h1 = jax.nn.relu(jnp.dot(w1_ref[...], pooled,
                             preferred_element_type=F32) + b1_ref[...])
    o_ref[...] = jnp.dot(w2_ref[...], h1,
                         preferred_element_type=F32) + b2_ref[...]


def kernel(x, W1, b1, We, be, Wl, bl, fc1w, fc1b, fc2w, fc2b,
           gate_w, gate_b, tw, tb):
    w1m = jnp.transpose(W1, (0, 2, 3, 1)).reshape(HID, 27)
    wem = jnp.transpose(We, (0, 1, 3, 4, 2)).reshape(NUM_LAYERS * HID, 9 * HID)
    wlm = jnp.transpose(Wl, (0, 2, 3, 1)).reshape(HID // 2, 9 * HID)
    gw8 = jnp.pad(gate_w, ((0, 4), (0, 0)))
    gb8 = jnp.pad(gate_b, (0, 4))[:, None]
    fc2p = jnp.pad(fc2w, ((0, 6), (0, 0)))
    fc2bp = jnp.pad(fc2b, (0, 6))[:, None]
    tbc = tb[:, None]

    xt = jnp.pad(x.transpose(1, 2, 3, 0), ((0, 0), (1, 1), (1, 1), (0, 0)))
    sl = [xt[:, ky:ky + 32:2, kx:kx + 32:2, :]
          for ky in range(3) for kx in range(3)]
    p1 = jnp.concatenate(sl, axis=0).reshape(27, 16 * 16 * BATCH)
    cb = 16 * 16 * BATCH // 8
    u = pl.pallas_call(
        _conv1_kernel,
        grid=(8,),
        in_specs=[pl.BlockSpec((HID, 27), lambda i: (0, 0)),
                  pl.BlockSpec((27, cb), lambda i: (0, i)),
                  pl.BlockSpec((HID, 1), lambda i: (0, 0))],
        out_specs=pl.BlockSpec((HID, cb), lambda i: (0, i)),
        out_shape=jax.ShapeDtypeStruct((HID, 16 * 16 * BATCH), F32),
    )(w1m, p1, b1[:, None])
    xfm = u.reshape(D_FLAT, BATCH)

    logits_out = []
    for _ in range(MAX_ROUTING):
        g8 = pl.pallas_call(
            _gate_kernel,
            in_specs=[pl.BlockSpec((D_FLAT, BATCH), lambda: (0, 0)),
                      pl.BlockSpec((8, D_FLAT), lambda: (0, 0)),
                      pl.BlockSpec((8, 1), lambda: (0, 0))],
            out_specs=pl.BlockSpec((8, BATCH), lambda: (0, 0)),
            out_shape=jax.ShapeDtypeStruct((8, BATCH), F32),
        )(xfm, gw8, gb8)
        logits_out.append(g8[NUM_LAYERS:, :].T)
        xp4 = jnp.pad(xfm.reshape(HID, 16, 16, BATCH),
                      ((0, 0), (1, 1), (1, 1), (0, 0)))
        fin = pl.pallas_call(
            _expert_kernel,
            grid=(4,),
            in_specs=[pl.BlockSpec((HID, 18, 18, BATCH),
                                   lambda i: (0, 0, 0, 0)),
                      pl.BlockSpec((NUM_LAYERS * HID, 9 * HID),
                                   lambda i: (0, 0)),
                      pl.BlockSpec((NUM_LAYERS * HID, 1), lambda i: (0, 0)),
                      pl.BlockSpec((8, BATCH), lambda i: (0, 0))],
            out_specs=pl.BlockSpec((HID, 4 * 16 * BATCH), lambda i: (0, i)),
            out_shape=jax.ShapeDtypeStruct((HID, 16 * 16 * BATCH), F32),
        )(xp4, wem, be.reshape(NUM_LAYERS * HID, 1), g8)
        fint = fin.reshape(D_FLAT, BATCH)
        rbk = 512
        xfm = pl.pallas_call(
            _mmt_kernel,
            grid=(D_FLAT // rbk,),
            in_specs=[pl.BlockSpec((rbk, D_FLAT), lambda j: (j, 0)),
                      pl.BlockSpec((D_FLAT, BATCH), lambda j: (0, 0)),
                      pl.BlockSpec((rbk, 1), lambda j: (j, 0))],
            out_specs=pl.BlockSpec((rbk, BATCH), lambda j: (j, 0)),
            out_shape=jax.ShapeDtypeStruct((D_FLAT, BATCH), F32),
        )(tw, fint, tbc)

    xp4 = jnp.pad(xfm.reshape(HID, 16, 16, BATCH),
                  ((0, 0), (1, 1), (1, 1), (0, 0)))
    sl = [xp4[:, ky:ky + 16:2, kx:kx + 16:2, :]
          for ky in range(3) for kx in range(3)]
    pt = jnp.concatenate(sl, axis=0)
    pt = pt.reshape(9 * HID, 4, 2, 4, 2, BATCH)
    pt = pt.transpose(0, 1, 3, 2, 4, 5).reshape(9 * HID, 64 * BATCH)
    outt = pl.pallas_call(
        _tail_kernel,
        in_specs=[pl.BlockSpec((9 * HID, 64 * BATCH), lambda: (0, 0)),
                  pl.BlockSpec((HID // 2, 9 * HID), lambda: (0, 0)),
                  pl.BlockSpec((HID // 2, 1), lambda: (0, 0)),
                  pl.BlockSpec((HID, 128), lambda: (0, 0)),
                  pl.BlockSpec((HID, 1), lambda: (0, 0)),
                  pl.BlockSpec((HID, HID), lambda: (0, 0)),
                  pl.BlockSpec((HID, 1), lambda: (0, 0))],
        out_specs=pl.BlockSpec((HID, BATCH), lambda: (0, 0)),
        out_shape=jax.ShapeDtypeStruct((HID, BATCH), F32),
    )(pt, wlm, bl[:, None], fc1w, fc1b[:, None], fc2p, fc2bp)
    out = outt[:10, :].T

    return (out, (logits_out[0], logits_out[1]))
